# baseline (speedup 1.0000x reference)
import functools

import jax
import jax.numpy as jnp
from jax import lax
from jax.experimental import pallas as pl
from jax.experimental.pallas import tpu as pltpu
from jax.experimental.pallas import tpu_sc as plsc

_N_EMB = 1024
_DIM = 64
_TILE = 4096
_N_ROWS = 32 * 1024
_NW = 32
_ROWS_PER_W = _N_ROWS // _NW
_CHUNK = 128
_NCH = _ROWS_PER_W // _CHUNK


def _argmin_body(x_ref, emb_ref, idx_ref, loss_ref):
    x = x_ref[...]
    emb = emb_ref[...]
    x2 = jnp.sum(x * x, axis=1, keepdims=True)
    e2 = jnp.sum(emb * emb, axis=1)
    dots = lax.dot_general(x, emb, (((1,), (1,)), ((), ())))
    dist = (x2 + e2[None, :]) - 2.0 * dots
    mins = jnp.min(dist, axis=1, keepdims=True)
    colf = lax.broadcasted_iota(jnp.int32, dist.shape, 1).astype(jnp.float32)
    idxf = jnp.min(jnp.where(dist == mins, colf, 16777216.0), axis=1)
    idx_ref[...] = idxf.astype(jnp.int32).reshape(idx_ref.shape)
    partial = jnp.sum(mins).reshape(1, 1)

    @pl.when(pl.program_id(0) == 0)
    def _init():
        loss_ref[...] = partial

    @pl.when(pl.program_id(0) != 0)
    def _acc():
        loss_ref[...] += partial


_NBUF = 2


def _make_sc_gather(n_rows):
    nch = n_rows // _NW // _CHUNK

    def _sc_gather_body(emb_hbm, idx_hbm, out_hbm, idx_v, rows_v, outc_v,
                        sem, wsem0, wsem1):
        wid = lax.axis_index("s") * 2 + lax.axis_index("c")
        base = wid * nch
        wsems = [wsem0, wsem1]
        pltpu.sync_copy(idx_hbm.at[pl.ds(base, nch)], idx_v)

        def _fire(j):
            return pltpu.async_copy(
                emb_hbm.at[idx_v.at[j]], rows_v.at[j % _NBUF], sem
            )

        copies = [_fire(j) for j in range(_NBUF)]
        writes = [None] * _NBUF
        for j in range(nch):
            b = j % _NBUF
            copies[b].wait()
            if writes[b] is not None:
                writes[b].wait()

            def _row(r, carry):
                for k in range(_DIM // 16):
                    outc_v[b, r, pl.ds(k * 16, 16)] = rows_v[b, r, pl.ds(k * 16, 16)]
                return carry

            lax.fori_loop(0, _CHUNK, _row, 0)
            if j + _NBUF < nch:
                copies[b] = _fire(j + _NBUF)
            writes[b] = pltpu.async_copy(
                outc_v.at[b],
                out_hbm.at[pl.ds((base + j) * _CHUNK, _CHUNK)],
                wsems[b],
            )
        for w in writes:
            if w is not None:
                w.wait()

    return functools.partial(
        pl.kernel,
        mesh=plsc.VectorSubcoreMesh(core_axis_name="c", subcore_axis_name="s"),
        out_type=jax.ShapeDtypeStruct((n_rows, _DIM), jnp.float32),
        scratch_types=[
            pltpu.VMEM((nch, _CHUNK), jnp.int32),
            pltpu.VMEM((_NBUF, _CHUNK, 2 * _DIM), jnp.float32),
            pltpu.VMEM((_NBUF, _CHUNK, _DIM), jnp.float32),
            pltpu.SemaphoreType.DMA,
            pltpu.SemaphoreType.DMA,
            pltpu.SemaphoreType.DMA,
        ],
        compiler_params=pltpu.CompilerParams(use_tc_tiling_on_sc=True),
    )(_sc_gather_body)


_sc_gather = _make_sc_gather(_N_ROWS)


def _argmin_call(flat_half, embeddings):
    n = flat_half.shape[0]
    grid = n // _TILE
    chunks_per_tile = _TILE // _CHUNK
    return pl.pallas_call(
        _argmin_body,
        grid=(grid,),
        in_specs=[
            pl.BlockSpec((_TILE, _DIM), lambda i: (i, 0)),
            pl.BlockSpec((_N_EMB, _DIM), lambda i: (0, 0)),
        ],
        out_specs=[
            pl.BlockSpec((chunks_per_tile, _CHUNK), lambda i: (i, 0)),
            pl.BlockSpec((1, 1), lambda i: (0, 0)),
        ],
        out_shape=[
            jax.ShapeDtypeStruct((n // _CHUNK, _CHUNK), jnp.int32),
            jax.ShapeDtypeStruct((1, 1), jnp.float32),
        ],
    )(flat_half, embeddings)


@jax.jit
def _vq(inputs, embeddings):
    flat = inputs.reshape(-1, _DIM)
    emb_p = jnp.pad(embeddings, ((0, 0), (0, _DIM)))
    idx, loss_sum = _argmin_call(flat, embeddings)
    q = _sc_gather(emb_p, idx)
    vq_loss = loss_sum[0, 0] * (2.0 / (_N_ROWS * _DIM))
    return q.reshape(inputs.shape), vq_loss


def kernel(inputs, embeddings):
    return _vq(inputs, embeddings)


# speedup vs baseline: 1.0485x; 1.0485x over previous
import functools

import jax
import jax.numpy as jnp
from jax import lax
from jax.experimental import pallas as pl
from jax.experimental.pallas import tpu as pltpu
from jax.experimental.pallas import tpu_sc as plsc

_N_EMB = 1024
_DIM = 64
_TILE = 4096
_N_ROWS = 32 * 1024
_NW = 32
_ROWS_PER_W = _N_ROWS // _NW
_CHUNK = 128
_NCH = _ROWS_PER_W // _CHUNK


def _argmin_body(x_ref, emb_ref, idx_ref, loss_ref):
    x = x_ref[...]
    emb = emb_ref[...]
    x2 = jnp.sum(x * x, axis=1, keepdims=True)
    e2 = jnp.sum(emb * emb, axis=1)
    dots2 = lax.dot_general(x, emb + emb, (((1,), (1,)), ((), ())))
    dist = (x2 + e2[None, :]) - dots2
    mins = jnp.min(dist, axis=1, keepdims=True)
    colf = lax.broadcasted_iota(jnp.int32, dist.shape, 1).astype(jnp.float32)
    idxf = jnp.min(jnp.where(dist == mins, colf, 16777216.0), axis=1)
    idx_ref[...] = idxf.astype(jnp.int32).reshape(idx_ref.shape)
    partial = jnp.sum(mins).reshape(1, 1)

    @pl.when(pl.program_id(0) == 0)
    def _init():
        loss_ref[...] = partial

    @pl.when(pl.program_id(0) != 0)
    def _acc():
        loss_ref[...] += partial


_NBUF = 2


def _make_sc_gather(n_rows):
    nch = n_rows // _NW // _CHUNK

    def _sc_gather_body(emb_hbm, idx_hbm, out_hbm, idx_v, rows_v, outc_v,
                        sem, wsem0, wsem1):
        wid = lax.axis_index("s") * 2 + lax.axis_index("c")
        base = wid * nch
        wsems = [wsem0, wsem1]
        pltpu.sync_copy(idx_hbm.at[pl.ds(base, nch)], idx_v)

        def _fire(j):
            return pltpu.async_copy(
                emb_hbm.at[idx_v.at[j]], rows_v.at[j % _NBUF], sem
            )

        copies = [_fire(j) for j in range(_NBUF)]
        writes = [None] * _NBUF
        for j in range(nch):
            b = j % _NBUF
            copies[b].wait()
            if writes[b] is not None:
                writes[b].wait()

            def _row(r, carry):
                for k in range(_DIM // 16):
                    outc_v[b, r, pl.ds(k * 16, 16)] = rows_v[b, r, pl.ds(k * 16, 16)]
                return carry

            lax.fori_loop(0, _CHUNK, _row, 0)
            if j + _NBUF < nch:
                copies[b] = _fire(j + _NBUF)
            writes[b] = pltpu.async_copy(
                outc_v.at[b],
                out_hbm.at[pl.ds((base + j) * _CHUNK, _CHUNK)],
                wsems[b],
            )
        for w in writes:
            if w is not None:
                w.wait()

    return functools.partial(
        pl.kernel,
        mesh=plsc.VectorSubcoreMesh(core_axis_name="c", subcore_axis_name="s"),
        out_type=jax.ShapeDtypeStruct((n_rows, _DIM), jnp.float32),
        scratch_types=[
            pltpu.VMEM((nch, _CHUNK), jnp.int32),
            pltpu.VMEM((_NBUF, _CHUNK, 2 * _DIM), jnp.float32),
            pltpu.VMEM((_NBUF, _CHUNK, _DIM), jnp.float32),
            pltpu.SemaphoreType.DMA,
            pltpu.SemaphoreType.DMA,
            pltpu.SemaphoreType.DMA,
        ],
        compiler_params=pltpu.CompilerParams(use_tc_tiling_on_sc=True),
    )(_sc_gather_body)


_sc_gather = _make_sc_gather(_N_ROWS)


def _argmin_call(flat_half, embeddings):
    n = flat_half.shape[0]
    grid = n // _TILE
    chunks_per_tile = _TILE // _CHUNK
    return pl.pallas_call(
        _argmin_body,
        grid=(grid,),
        in_specs=[
            pl.BlockSpec((_TILE, _DIM), lambda i: (i, 0)),
            pl.BlockSpec((_N_EMB, _DIM), lambda i: (0, 0)),
        ],
        out_specs=[
            pl.BlockSpec((chunks_per_tile, _CHUNK), lambda i: (i, 0)),
            pl.BlockSpec((1, 1), lambda i: (0, 0)),
        ],
        out_shape=[
            jax.ShapeDtypeStruct((n // _CHUNK, _CHUNK), jnp.int32),
            jax.ShapeDtypeStruct((1, 1), jnp.float32),
        ],
    )(flat_half, embeddings)


@jax.jit
def _vq(inputs, embeddings):
    flat = inputs.reshape(-1, _DIM)
    emb_p = jnp.pad(embeddings, ((0, 0), (0, _DIM)))
    idx, loss_sum = _argmin_call(flat, embeddings)
    q = _sc_gather(emb_p, idx)
    vq_loss = loss_sum[0, 0] * (2.0 / (_N_ROWS * _DIM))
    return q.reshape(inputs.shape), vq_loss


def kernel(inputs, embeddings):
    return _vq(inputs, embeddings)


# speedup vs baseline: 1.0643x; 1.0150x over previous
import functools

import jax
import jax.numpy as jnp
from jax import lax
from jax.experimental import pallas as pl
from jax.experimental.pallas import tpu as pltpu
from jax.experimental.pallas import tpu_sc as plsc

_N_EMB = 1024
_DIM = 64
_TILE = 4096
_N_ROWS = 32 * 1024
_NW = 32
_ROWS_PER_W = _N_ROWS // _NW
_CHUNK = 128
_NCH = _ROWS_PER_W // _CHUNK


def _argmin_body(x_ref, emb_ref, idx_ref, loss_ref, embp_ref):
    x = x_ref[...]
    emb = emb_ref[...]
    x2 = jnp.sum(x * x, axis=1, keepdims=True)
    e2 = jnp.sum(emb * emb, axis=1)
    dots2 = lax.dot_general(x, emb + emb, (((1,), (1,)), ((), ())))
    dist = (x2 + e2[None, :]) - dots2
    mins = jnp.min(dist, axis=1, keepdims=True)
    colf = lax.broadcasted_iota(jnp.int32, dist.shape, 1).astype(jnp.float32)
    idxf = jnp.min(jnp.where(dist == mins, colf, 16777216.0), axis=1)
    idx_ref[...] = idxf.astype(jnp.int32).reshape(idx_ref.shape)
    partial = jnp.sum(mins).reshape(1, 1)

    @pl.when(pl.program_id(0) == 0)
    def _init():
        loss_ref[...] = partial
        embp_ref[...] = jnp.concatenate(
            [emb, jnp.zeros((_N_EMB, _DIM), jnp.float32)], axis=1
        )

    @pl.when(pl.program_id(0) != 0)
    def _acc():
        loss_ref[...] += partial


_NBUF = 2


def _make_sc_gather(n_rows):
    nch = n_rows // _NW // _CHUNK

    def _sc_gather_body(emb_hbm, idx_hbm, out_hbm, idx_v, rows_v, outc_v,
                        sem, wsem0, wsem1):
        wid = lax.axis_index("s") * 2 + lax.axis_index("c")
        base = wid * nch
        wsems = [wsem0, wsem1]
        pltpu.sync_copy(idx_hbm.at[pl.ds(base, nch)], idx_v)

        def _fire(j):
            return pltpu.async_copy(
                emb_hbm.at[idx_v.at[j]], rows_v.at[j % _NBUF], sem
            )

        copies = [_fire(j) for j in range(_NBUF)]
        writes = [None] * _NBUF
        for j in range(nch):
            b = j % _NBUF
            copies[b].wait()
            if writes[b] is not None:
                writes[b].wait()

            def _row(r, carry):
                for k in range(_DIM // 16):
                    outc_v[b, r, pl.ds(k * 16, 16)] = rows_v[b, r, pl.ds(k * 16, 16)]
                return carry

            lax.fori_loop(0, _CHUNK, _row, 0)
            if j + _NBUF < nch:
                copies[b] = _fire(j + _NBUF)
            writes[b] = pltpu.async_copy(
                outc_v.at[b],
                out_hbm.at[pl.ds((base + j) * _CHUNK, _CHUNK)],
                wsems[b],
            )
        for w in writes:
            if w is not None:
                w.wait()

    return functools.partial(
        pl.kernel,
        mesh=plsc.VectorSubcoreMesh(core_axis_name="c", subcore_axis_name="s"),
        out_type=jax.ShapeDtypeStruct((n_rows, _DIM), jnp.float32),
        scratch_types=[
            pltpu.VMEM((nch, _CHUNK), jnp.int32),
            pltpu.VMEM((_NBUF, _CHUNK, 2 * _DIM), jnp.float32),
            pltpu.VMEM((_NBUF, _CHUNK, _DIM), jnp.float32),
            pltpu.SemaphoreType.DMA,
            pltpu.SemaphoreType.DMA,
            pltpu.SemaphoreType.DMA,
        ],
        compiler_params=pltpu.CompilerParams(use_tc_tiling_on_sc=True),
    )(_sc_gather_body)


_sc_gather = _make_sc_gather(_N_ROWS)


def _argmin_call(flat_half, embeddings):
    n = flat_half.shape[0]
    grid = n // _TILE
    chunks_per_tile = _TILE // _CHUNK
    return pl.pallas_call(
        _argmin_body,
        grid=(grid,),
        in_specs=[
            pl.BlockSpec((_TILE, _DIM), lambda i: (i, 0)),
            pl.BlockSpec((_N_EMB, _DIM), lambda i: (0, 0)),
        ],
        out_specs=[
            pl.BlockSpec((chunks_per_tile, _CHUNK), lambda i: (i, 0)),
            pl.BlockSpec((1, 1), lambda i: (0, 0)),
            pl.BlockSpec((_N_EMB, 2 * _DIM), lambda i: (0, 0)),
        ],
        out_shape=[
            jax.ShapeDtypeStruct((n // _CHUNK, _CHUNK), jnp.int32),
            jax.ShapeDtypeStruct((1, 1), jnp.float32),
            jax.ShapeDtypeStruct((_N_EMB, 2 * _DIM), jnp.float32),
        ],
    )(flat_half, embeddings)


@jax.jit
def _vq(inputs, embeddings):
    flat = inputs.reshape(-1, _DIM)
    idx, loss_sum, emb_p = _argmin_call(flat, embeddings)
    q = _sc_gather(emb_p, idx)
    vq_loss = loss_sum[0, 0] * (2.0 / (_N_ROWS * _DIM))
    return q.reshape(inputs.shape), vq_loss


def kernel(inputs, embeddings):
    return _vq(inputs, embeddings)
